# baseline (speedup 1.0000x reference)
import jax
import jax.numpy as jnp
from jax import lax
from jax.experimental import pallas as pl

_SLABS = 128
_ROWS_IN = 1792
_ROWS_DATA = 1664


def _demap_body(in_ref, data_ref, pil_ref):
    half = pl.program_id(1)

    rI = lax.broadcasted_iota(jnp.int32, (128, 128), 0)
    cI = lax.broadcasted_iota(jnp.int32, (128, 128), 1)
    b8R = (rI % 32) // 4
    idxO = 2 * (cI % 64) + 1
    maskO = (cI // 64) == ((rI % 8) // 4)
    maskP = (cI // 16) == b8R

    @pl.when(half == 0)
    def _():
        data_ref[0, 0:256] = in_ref[0, 0:256]
        data_ref[0, 320:832] = in_ref[0, 384:896]

    @pl.when(half == 1)
    def _():
        data_ref[0, 0:512] = in_ref[0, 0:512]
        data_ref[0, 576:832] = in_ref[0, 640:896]

    sym_row = jnp.where(half == 0, 256, 512)
    data_row = jnp.where(half == 0, 256, 512)
    ys = in_ref[0, pl.ds(sym_row, 128)]
    g = jnp.take_along_axis(ys, idxO, axis=1)
    xo = jnp.where(maskO, g, 0.0)
    w = xo.reshape(16, 2, 4, 128).sum(axis=1).reshape(64, 128)
    data_ref[0, pl.ds(data_row, 64)] = w
    for i in range(4):
        idxP = 8 * (cI % 16) + 2 * i
        gp = jnp.take_along_axis(ys, idxP, axis=1)
        xp = jnp.where(maskP, gp, 0.0)
        zs = xp.reshape(4, 8, 4, 128).sum(axis=1).reshape(16, 128)
        pil_ref[0, pl.ds(32 * i + 16 * half, 16)] = zs


@jax.jit
def kernel(inputs):
    b, tx, s, re, n = inputs.shape
    y = (inputs.reshape(b, tx, s, 448, 128, n)
         .transpose(0, 1, 2, 3, 5, 4)
         .reshape(_SLABS, _ROWS_IN, 128))
    data_y, z = pl.pallas_call(
        _demap_body,
        grid=(_SLABS, 2),
        in_specs=[pl.BlockSpec((1, _ROWS_IN // 2, 128),
                               lambda i, j: (i, j, 0))],
        out_specs=[
            pl.BlockSpec((1, _ROWS_DATA // 2, 128), lambda i, j: (i, j, 0)),
            pl.BlockSpec((1, 128, 128), lambda i, j: (i, 0, 0)),
        ],
        out_shape=[
            jax.ShapeDtypeStruct((_SLABS, _ROWS_DATA, 128), jnp.float32),
            jax.ShapeDtypeStruct((_SLABS, 128, 128), jnp.float32),
        ],
    )(y)
    data = (data_y.reshape(b, tx, s, 416, n, 128)
            .transpose(0, 1, 2, 3, 5, 4)
            .reshape(b, tx, s, 53248, n))
    pilots = (z.reshape(b, tx, s, 4, 8, 4, 128)
              .transpose(0, 1, 2, 4, 6, 3, 5)
              .reshape(b, tx, s, 1024, n, n))
    return (data, pilots)


# speedup vs baseline: 1.4824x; 1.4824x over previous
import jax
import jax.numpy as jnp
from jax import lax
from jax.experimental import pallas as pl

_SLABS = 128
_ROWS_IN = 1792
_ROWS_DATA = 1664


def _demap_body(in_ref, data_ref, pil_ref):
    data_ref[0, 0:256] = in_ref[0, 0:256]
    data_ref[0, 320:1344] = in_ref[0, 384:1408]
    data_ref[0, 1408:1664] = in_ref[0, 1536:1792]

    rI = lax.broadcasted_iota(jnp.int32, (128, 128), 0)
    cI = lax.broadcasted_iota(jnp.int32, (128, 128), 1)
    b8R = (rI % 32) // 4
    idxO = 2 * (cI % 64) + 1
    maskO = (cI // 64) == ((rI % 8) // 4)
    maskP = (cI // 16) == b8R

    for sym_row, pil_half, data_row in ((256, 0, 256), (1408, 1, 1344)):
        ys = in_ref[0, sym_row:sym_row + 128]
        g = jnp.take_along_axis(ys, idxO, axis=1)
        xo = jnp.where(maskO, g, 0.0)
        w = xo.reshape(16, 2, 4, 128).sum(axis=1).reshape(64, 128)
        data_ref[0, data_row:data_row + 64] = w
        for i in range(4):
            idxP = 8 * (cI % 16) + 2 * i
            gp = jnp.take_along_axis(ys, idxP, axis=1)
            xp = jnp.where(maskP, gp, 0.0)
            zs = xp.reshape(4, 8, 4, 128).sum(axis=1).reshape(16, 128)
            r0 = 32 * i + 16 * pil_half
            pil_ref[0, r0:r0 + 16] = zs


@jax.jit
def kernel(inputs):
    b, tx, s, re, n = inputs.shape
    y = (inputs.reshape(b, tx, s, 448, 128, n)
         .transpose(0, 1, 2, 3, 5, 4)
         .reshape(_SLABS, _ROWS_IN, 128))
    data_y, z = pl.pallas_call(
        _demap_body,
        grid=(_SLABS,),
        in_specs=[pl.BlockSpec((1, _ROWS_IN, 128), lambda i: (i, 0, 0))],
        out_specs=[
            pl.BlockSpec((1, _ROWS_DATA, 128), lambda i: (i, 0, 0)),
            pl.BlockSpec((1, 128, 128), lambda i: (i, 0, 0)),
        ],
        out_shape=[
            jax.ShapeDtypeStruct((_SLABS, _ROWS_DATA, 128), jnp.float32),
            jax.ShapeDtypeStruct((_SLABS, 128, 128), jnp.float32),
        ],
    )(y)
    data = (data_y.reshape(b, tx, s, 416, n, 128)
            .transpose(0, 1, 2, 3, 5, 4)
            .reshape(b, tx, s, 53248, n))
    pilots = (z.reshape(b, tx, s, 4, 8, 4, 128)
              .transpose(0, 1, 2, 4, 6, 3, 5)
              .reshape(b, tx, s, 1024, n, n))
    return (data, pilots)
